# baseline (speedup 1.0000x reference)
import functools

import jax
import jax.numpy as jnp
import numpy as np
from jax import lax
from jax.experimental import pallas as pl
from jax.experimental.pallas import tpu as pltpu
from jax.experimental.pallas import tpu_sc as plsc

S, D, H, E, F = 2048, 768, 12, 64, 64
DH = D // H
NB = S // 128
ND = 2 * NB - 1
NP = 2 * S
NT = 96
NW = 32

_INTERPRET = False


def _pc(body, grid, in_specs, out_specs, out_shape, scratch_shapes=()):
    return pl.pallas_call(
        body,
        grid=grid,
        in_specs=in_specs,
        out_specs=out_specs,
        out_shape=out_shape,
        scratch_shapes=list(scratch_shapes),
        interpret=_INTERPRET,
    )


def _ln(x, g, b):
    m = jnp.mean(x, axis=-1, keepdims=True)
    v = jnp.mean((x - m) ** 2, axis=-1, keepdims=True)
    return (x - m) * jax.lax.rsqrt(v + 1e-5) * g + b


def _k1_body(src_ref, g_ref, b_ref, wq_ref, wk_ref, wv_ref, q_ref, k_ref, v_ref):
    x2 = _ln(src_ref[...], g_ref[...], b_ref[...]).astype(jnp.bfloat16)
    q = lax.dot(x2, wq_ref[...], preferred_element_type=jnp.float32) * 0.125
    k = lax.dot(x2, wk_ref[...], preferred_element_type=jnp.float32)
    v = lax.dot(x2, wv_ref[...], preferred_element_type=jnp.float32)
    qb, kb, vb = q.astype(jnp.bfloat16), k.astype(jnp.bfloat16), v.astype(jnp.bfloat16)
    for h in range(H):
        sl = slice(h * DH, (h + 1) * DH)
        q_ref[h] = qb[:, sl]
        k_ref[h] = kb[:, sl]
        v_ref[h] = vb[:, sl]


def _k1(src, ln1_g, ln1_b, wq, wk, wv):
    spec_w = pl.BlockSpec((D, D), lambda i: (0, 0))
    spec_v = pl.BlockSpec((1, D), lambda i: (0, 0))
    out_spec = pl.BlockSpec((H, 128, DH), lambda i: (0, i, 0))
    return _pc(
        _k1_body,
        grid=(NB,),
        in_specs=[pl.BlockSpec((128, D), lambda i: (i, 0)), spec_v, spec_v,
                  spec_w, spec_w, spec_w],
        out_specs=[out_spec] * 3,
        out_shape=[jax.ShapeDtypeStruct((H, S, DH), jnp.bfloat16)] * 3,
    )(src, ln1_g.reshape(1, D), ln1_b.reshape(1, D),
      wq.astype(jnp.bfloat16), wk.astype(jnp.bfloat16), wv.astype(jnp.bfloat16))


def _k2_body(bl_ref, br_ref, e_ref, out_ref):
    sm = jnp.concatenate([bl_ref[...], br_ref[...]], axis=1).astype(jnp.bfloat16)
    mm = lax.dot(sm, e_ref[...], preferred_element_type=jnp.float32)
    out_ref[0] = mm.astype(jnp.bfloat16)


def _k2(rel_bias):
    rb = jnp.pad(rel_bias, ((0, 0), (0, 4096 - (2 * S - 1))))
    ab = np.arange(128 * 128)
    a, b = ab // 128, ab % 128
    c = np.arange(256)
    e_mat = (c[:, None] == (b - a + 127)[None, :]).astype(np.float32)
    e_mat = jnp.asarray(e_mat, dtype=jnp.bfloat16)
    t3 = _pc(
        _k2_body,
        grid=(ND,),
        in_specs=[pl.BlockSpec((H, 128), lambda d: (0, d)),
                  pl.BlockSpec((H, 128), lambda d: (0, d + 1)),
                  pl.BlockSpec((256, 128 * 128), lambda d: (0, 0))],
        out_specs=pl.BlockSpec((1, H, 128 * 128), lambda d: (d, 0, 0)),
        out_shape=jax.ShapeDtypeStruct((ND, H, 128 * 128), jnp.bfloat16),
    )(rb, rb, e_mat)
    return t3.reshape(ND, H, 128, 128)


def _k3_body(q_ref, k_ref, v_ref, t_ref, o_ref):
    h = pl.program_id(0)
    i = pl.program_id(1)
    q = q_ref[0]
    k = k_ref[0]
    s = lax.dot_general(q, k, (((1,), (1,)), ((), ())),
                        preferred_element_type=jnp.float32)
    patt = jnp.concatenate(
        [t_ref[j - i + (NB - 1), h].astype(jnp.float32) for j in range(NB)], axis=1)
    s = s + patt
    m = jnp.max(s, axis=1, keepdims=True)
    p = jnp.exp(s - m)
    l = jnp.sum(p, axis=1, keepdims=True)
    att = (p / l).astype(jnp.bfloat16)
    o = lax.dot(att, v_ref[0], preferred_element_type=jnp.float32)
    o_ref[0] = o.astype(jnp.bfloat16)


def _k3(q, k, v, t4):
    return _pc(
        _k3_body,
        grid=(H, NB),
        in_specs=[pl.BlockSpec((1, 128, DH), lambda h, i: (h, i, 0)),
                  pl.BlockSpec((1, S, DH), lambda h, i: (h, 0, 0)),
                  pl.BlockSpec((1, S, DH), lambda h, i: (h, 0, 0)),
                  pl.BlockSpec((ND, H, 128, 128), lambda h, i: (0, 0, 0, 0))],
        out_specs=pl.BlockSpec((1, 128, DH), lambda h, i: (h, i, 0)),
        out_shape=jax.ShapeDtypeStruct((H, S, DH), jnp.bfloat16),
    )(q, k, v, t4)


def _k4_body(att_ref, wo_ref, src_ref, out_ref):
    cat = jnp.concatenate([att_ref[h] for h in range(H)], axis=1)
    o = lax.dot(cat, wo_ref[...], preferred_element_type=jnp.float32)
    out_ref[...] = src_ref[...] + o


def _k4(att, wo, src):
    return _pc(
        _k4_body,
        grid=(NB,),
        in_specs=[pl.BlockSpec((H, 128, DH), lambda i: (0, i, 0)),
                  pl.BlockSpec((D, D), lambda i: (0, 0)),
                  pl.BlockSpec((128, D), lambda i: (i, 0))],
        out_specs=pl.BlockSpec((128, D), lambda i: (i, 0)),
        out_shape=jax.ShapeDtypeStruct((S, D), jnp.float32),
    )(att, wo.astype(jnp.bfloat16), src)


def _k5_body(src_ref, g_ref, b_ref, es_ref, x3_ref, gates_ref, idx_ref):
    x3 = _ln(src_ref[...], g_ref[...], b_ref[...])
    x3_ref[...] = x3
    logits = lax.dot(x3, es_ref[...], preferred_element_type=jnp.float32)
    sel = jax.nn.sigmoid(logits)
    iota = lax.broadcasted_iota(jnp.int32, sel.shape, 1)
    m1 = jnp.max(sel, axis=1, keepdims=True)
    i1 = jnp.min(jnp.where(sel == m1, iota, E), axis=1, keepdims=True)
    masked = jnp.where(iota == i1, -1.0, sel)
    m2 = jnp.max(masked, axis=1, keepdims=True)
    i2 = jnp.min(jnp.where(masked == m2, iota, E), axis=1, keepdims=True)
    gates_ref[...] = jnp.concatenate([m1, m2], axis=1)
    idx_ref[...] = jnp.concatenate([i1, i2], axis=1)


def _k5(src2, ln2_g, ln2_b, expert_sel):
    spec_v = pl.BlockSpec((1, D), lambda i: (0, 0))
    return _pc(
        _k5_body,
        grid=(NB,),
        in_specs=[pl.BlockSpec((128, D), lambda i: (i, 0)), spec_v, spec_v,
                  pl.BlockSpec((D, E), lambda i: (0, 0))],
        out_specs=[pl.BlockSpec((128, D), lambda i: (i, 0)),
                   pl.BlockSpec((128, 2), lambda i: (i, 0)),
                   pl.BlockSpec((128, 2), lambda i: (i, 0))],
        out_shape=[jax.ShapeDtypeStruct((S, D), jnp.float32),
                   jax.ShapeDtypeStruct((S, 2), jnp.float32),
                   jax.ShapeDtypeStruct((S, 2), jnp.int32)],
    )(src2, ln2_g.reshape(1, D), ln2_b.reshape(1, D), expert_sel)


def _k6s_body(idx_ref, lt_ref, sl_ref, pos_ref, eot_ref):
    def pass1(pb, cnt):
        idxb = idx_ref[pl.ds(pb * 128, 128), :]
        oh = (idxb == lax.broadcasted_iota(jnp.int32, (128, E), 1))
        ohb = oh.astype(jnp.bfloat16)
        cums = lax.dot(lt_ref[...], ohb, preferred_element_type=jnp.float32)
        rank = jnp.sum(oh.astype(jnp.float32) * (cums + cnt), axis=1,
                       keepdims=True)
        pos_ref[pl.ds(pb * 128, 128), :] = rank.astype(jnp.int32)
        return cnt + jnp.sum(oh.astype(jnp.float32), axis=0, keepdims=True)

    cnt = lax.fori_loop(0, NP // 128, pass1, jnp.zeros((1, E), jnp.float32))
    ntiles = jnp.ceil(cnt * (1.0 / 128.0))
    tilestart = lax.dot(ntiles.astype(jnp.bfloat16), sl_ref[...],
                        preferred_element_type=jnp.float32)
    startpad = tilestart * 128.0

    def pass2(pb, _):
        idxb = idx_ref[pl.ds(pb * 128, 128), :]
        oh = (idxb == lax.broadcasted_iota(jnp.int32, (128, E), 1))
        add = jnp.sum(oh.astype(jnp.float32) * startpad, axis=1,
                      keepdims=True).astype(jnp.int32)
        pos_ref[pl.ds(pb * 128, 128), :] = pos_ref[pl.ds(pb * 128, 128), :] + add
        return 0

    lax.fori_loop(0, NP // 128, pass2, 0)
    ts_i = tilestart.astype(jnp.int32)
    tio = lax.broadcasted_iota(jnp.int32, (NT, E), 0)
    eot_ref[...] = jnp.sum((tio >= ts_i).astype(jnp.int32), axis=1,
                           keepdims=True) - 1


def _k6_sort(idxp):
    lt = jnp.asarray(np.tril(np.ones((128, 128), np.float32), -1),
                     dtype=jnp.bfloat16)
    sl = jnp.asarray(np.triu(np.ones((E, E), np.float32), 1),
                     dtype=jnp.bfloat16)
    return _pc(
        _k6s_body,
        grid=(1,),
        in_specs=[pl.BlockSpec((NP, 1), lambda i: (0, 0)),
                  pl.BlockSpec((128, 128), lambda i: (0, 0)),
                  pl.BlockSpec((E, E), lambda i: (0, 0))],
        out_specs=[pl.BlockSpec((NP, 1), lambda i: (0, 0)),
                   pl.BlockSpec((NT, 1), lambda i: (0, 0))],
        out_shape=[jax.ShapeDtypeStruct((NP, 1), jnp.int32),
                   jax.ShapeDtypeStruct((NT, 1), jnp.int32)],
    )(idxp, lt, sl)


def _k7_body(eot_sref, xg_ref, k_ref, v_ref, yg_ref):
    x = xg_ref[...].astype(jnp.bfloat16)
    hid = jax.nn.relu(lax.dot(x, k_ref[0], preferred_element_type=jnp.float32))
    yg_ref[...] = lax.dot(hid.astype(jnp.bfloat16), v_ref[0],
                          preferred_element_type=jnp.float32)


def _k7_group(xg, keys_bf, values_bf, eot_flat):
    grid_spec = pltpu.PrefetchScalarGridSpec(
        num_scalar_prefetch=1,
        grid=(NT,),
        in_specs=[pl.BlockSpec((128, D), lambda t, e: (t, 0)),
                  pl.BlockSpec((1, D, F), lambda t, e: (e[t], 0, 0)),
                  pl.BlockSpec((1, F, D), lambda t, e: (e[t], 0, 0))],
        out_specs=pl.BlockSpec((128, D), lambda t, e: (t, 0)),
    )
    return pl.pallas_call(
        _k7_body,
        grid_spec=grid_spec,
        out_shape=jax.ShapeDtypeStruct((NT * 128, D), jnp.float32),
        interpret=_INTERPRET,
    )(eot_flat, xg, keys_bf, values_bf)


def _sc_mesh():
    return plsc.VectorSubcoreMesh(core_axis_name="c", subcore_axis_name="s",
                                  num_cores=2)


def _k8_dispatch(x3, pos2, tok2):
    @functools.partial(
        pl.kernel, mesh=_sc_mesh(),
        out_type=jax.ShapeDtypeStruct((NT * 128, D), jnp.float32),
        scratch_types=[pltpu.VMEM((2, 64), jnp.int32),
                       pltpu.VMEM((2, 64), jnp.int32),
                       pltpu.VMEM((64, D), jnp.float32),
                       pltpu.SemaphoreType.DMA],
    )
    def k(x3_hbm, pos_hbm, tok_hbm, xg_hbm, tok_v, pos_v, rows_v, sem):
        wid = lax.axis_index("s") * 2 + lax.axis_index("c")
        pltpu.sync_copy(tok_hbm.at[pl.ds(wid * 2, 2)], tok_v)
        pltpu.sync_copy(pos_hbm.at[pl.ds(wid * 2, 2)], pos_v)
        for c in range(2):
            pltpu.async_copy(x3_hbm.at[tok_v.at[c]], rows_v, sem).wait()
            pltpu.async_copy(rows_v, xg_hbm.at[pos_v.at[c]], sem).wait()

    return k(x3, pos2, tok2)


def _k9_combine(yg, p0, p1, g0, g1, src2):
    @functools.partial(
        pl.kernel, mesh=_sc_mesh(),
        out_type=jax.ShapeDtypeStruct((S, D), jnp.float32),
        scratch_types=[pltpu.VMEM((4, 16), jnp.int32),
                       pltpu.VMEM((4, 16), jnp.int32),
                       pltpu.VMEM((4, 16), jnp.float32),
                       pltpu.VMEM((4, 16), jnp.float32),
                       pltpu.VMEM((16, D), jnp.float32),
                       pltpu.VMEM((16, D), jnp.float32),
                       pltpu.VMEM((16, D), jnp.float32),
                       pltpu.SemaphoreType.DMA],
    )
    def k(yg_hbm, p0_hbm, p1_hbm, g0_hbm, g1_hbm, src_hbm, out_hbm,
          p0v, p1v, g0v, g1v, r0v, r1v, sv, sem):
        wid = lax.axis_index("s") * 2 + lax.axis_index("c")
        pltpu.sync_copy(p0_hbm.at[pl.ds(wid * 4, 4)], p0v)
        pltpu.sync_copy(p1_hbm.at[pl.ds(wid * 4, 4)], p1v)
        pltpu.sync_copy(g0_hbm.at[pl.ds(wid * 4, 4)], g0v)
        pltpu.sync_copy(g1_hbm.at[pl.ds(wid * 4, 4)], g1v)
        for c in range(4):
            pltpu.async_copy(yg_hbm.at[p0v.at[c]], r0v, sem).wait()
            pltpu.async_copy(yg_hbm.at[p1v.at[c]], r1v, sem).wait()
            base = wid * 64 + c * 16
            pltpu.sync_copy(src_hbm.at[pl.ds(base, 16)], sv)
            g0row = g0v[c, :]
            g1row = g1v[c, :]
            for i in range(16):
                a = g0row[i]
                b = g1row[i]

                def col(j, _2, i=i, a=a, b=b):
                    sl = pl.ds(j * 16, 16)
                    sv[i, sl] = sv[i, sl] + a * r0v[i, sl] + b * r1v[i, sl]
                    return 0

                lax.fori_loop(0, D // 16, col, 0)
            pltpu.sync_copy(sv, out_hbm.at[pl.ds(base, 16)])

    return k(yg, p0, p1, g0, g1, src2)


_TOK2 = np.repeat(np.arange(S, dtype=np.int32), 2).reshape(NW * 2, 64)


def kernel(src, ln1_g, ln1_b, ln2_g, ln2_b, Wq, Wk, Wv, Wo, rel_bias,
           expert_sel, keys, values):
    src2d = src.reshape(S, D)
    q, k, v = _k1(src2d, ln1_g, ln1_b, Wq, Wk, Wv)
    t4 = _k2(rel_bias)
    att = _k3(q, k, v, t4)
    src2 = _k4(att, Wo, src2d)
    x3, gates2, idx2 = _k5(src2, ln2_g, ln2_b, expert_sel)
    pos, eot = _k6_sort(idx2.reshape(NP, 1))
    xg = _k8_dispatch(x3, pos.reshape(NW * 2, 64), jnp.asarray(_TOK2))
    yg = _k7_group(xg, keys.astype(jnp.bfloat16), values.astype(jnp.bfloat16),
                   eot.reshape(NT))
    posT = pos.reshape(S, 2)
    out = _k9_combine(yg,
                      posT[:, 0].reshape(NW * 4, 16),
                      posT[:, 1].reshape(NW * 4, 16),
                      gates2[:, 0].reshape(NW * 4, 16),
                      gates2[:, 1].reshape(NW * 4, 16),
                      src2)
    return out.reshape(1, S, D)
